# Initial kernel scaffold; baseline (speedup 1.0000x reference)
import jax
import jax.numpy as jnp
from jax.experimental import pallas as pl


def kernel(x, emb_table, W, b):
    raise NotImplementedError("write your pallas kernel here")


# speedup vs baseline: 3.9247x; 3.9247x over previous
import functools

import jax
import jax.numpy as jnp
from jax import lax
from jax.experimental import pallas as pl
from jax.experimental.pallas import tpu as pltpu
from jax.experimental.pallas import tpu_sc as plsc

B = 4096
K = 8
D = 16
V = 1000
NC, NS = 2, 16
NW = NC * NS
RPW = B // NW
NG = RPW // 16
IPW = RPW * K


def _sc_kernel(x_hbm, tbl_hbm, w_hbm, b_hbm, out_hbm,
               idx_v, tbl_v, w_v, b_v, out_v):
    wid = lax.axis_index("s") * NC + lax.axis_index("c")

    pltpu.sync_copy(x_hbm.at[pl.ds(wid * IPW, IPW)], idx_v)
    pltpu.sync_copy(tbl_hbm, tbl_v)
    pltpu.sync_copy(w_hbm, w_v)
    pltpu.sync_copy(b_hbm, b_v)

    lane8 = lax.iota(jnp.int32, 16) * 8

    def group(g, carry):
        acc = b_v[...]
        base = g * 128
        for j in range(K):
            xv = plsc.load_gather(idx_v, [lane8 + (base + j)])
            xv16 = xv * D
            wj = w_v[pl.ds(j * D, D)]
            for f in range(D):
                val = plsc.load_gather(tbl_v, [xv16 + f])
                acc = acc + val * wj[f]
        e = jnp.exp(acc - acc)
        out_v[pl.ds(g * 16, 16)] = e / e
        return carry

    lax.fori_loop(0, NG, group, 0)
    pltpu.sync_copy(out_v, out_hbm.at[pl.ds(wid * RPW, RPW)])


@jax.jit
def _run(x_flat, tbl_flat, w_flat, b16):
    mesh = plsc.VectorSubcoreMesh(core_axis_name="c", subcore_axis_name="s")
    kfn = functools.partial(
        pl.kernel,
        mesh=mesh,
        compiler_params=pltpu.CompilerParams(needs_layout_passes=False),
        out_type=jax.ShapeDtypeStruct((B,), jnp.float32),
        scratch_types=[
            pltpu.VMEM((IPW,), jnp.int32),
            pltpu.VMEM((V * D,), jnp.float32),
            pltpu.VMEM((K * D,), jnp.float32),
            pltpu.VMEM((16,), jnp.float32),
            pltpu.VMEM((RPW,), jnp.float32),
        ],
    )(_sc_kernel)
    return kfn(x_flat, tbl_flat, w_flat, b16)


def kernel(x, emb_table, W, b):
    x_flat = x.reshape(B * K)
    tbl_flat = emb_table.reshape(V * D)
    w_flat = W.reshape(K * D)
    b16 = jnp.broadcast_to(b, (16,)).astype(jnp.float32)
    out = _run(x_flat, tbl_flat, w_flat, b16)
    return out.reshape(B, 1)


# speedup vs baseline: 4.4480x; 1.1333x over previous
import functools

import jax
import jax.numpy as jnp
from jax import lax
from jax.experimental import pallas as pl
from jax.experimental.pallas import tpu as pltpu
from jax.experimental.pallas import tpu_sc as plsc

B = 4096
K = 8
D = 16
V = 1000
VP = 1024
NC, NS = 2, 16
NW = NC * NS
RPW = B // NW
NG = RPW // 16
IPW = RPW * K
VPS = VP // NS


def _sc_kernel(x_hbm, tbl_hbm, w_hbm, b_hbm, out_hbm,
               idx_v, tbl_v, w_v, b_v, tloc_v, t_v, out_v, t_sh):
    cid = lax.axis_index("c")
    sid = lax.axis_index("s")
    wid = sid * NC + cid

    pltpu.sync_copy(x_hbm.at[pl.ds(wid * IPW, IPW)], idx_v)
    pltpu.sync_copy(tbl_hbm.at[pl.ds(sid * VPS * D, VPS * D)], tbl_v)
    pltpu.sync_copy(w_hbm, w_v)
    pltpu.sync_copy(b_hbm, b_v)

    lane = lax.iota(jnp.int32, 16)
    lane8 = lane * 8
    lane16 = lane * D

    wj = [w_v[pl.ds(j * D, D)] for j in range(K)]

    zero = jnp.zeros((16,), jnp.float32)
    for r in range(VPS // 16):
        accs = [zero] * K
        for f in range(D):
            col = plsc.load_gather(tbl_v, [lane16 + (r * 16 * D + f)])
            for j in range(K):
                accs[j] = accs[j] + col * wj[j][f]
        for j in range(K):
            plsc.store_scatter(tloc_v, [lane8 + (r * 128 + j)], accs[j])

    pltpu.sync_copy(tloc_v, t_sh.at[pl.ds(sid * (VPS * K), VPS * K)])
    plsc.subcore_barrier()
    pltpu.sync_copy(t_sh, t_v)

    for g in range(NG):
        acc = b_v[...]
        for j in range(K):
            xv = plsc.load_gather(idx_v, [lane8 + (g * 128 + j)])
            val = plsc.load_gather(t_v, [xv * K + j])
            acc = acc + val
        e = jnp.exp(acc - acc)
        out_v[pl.ds(g * 16, 16)] = e / e

    pltpu.sync_copy(out_v, out_hbm.at[pl.ds(wid * RPW, RPW)])


@jax.jit
def _run(x_flat, tbl_flat, w_flat, b16):
    mesh = plsc.VectorSubcoreMesh(core_axis_name="c", subcore_axis_name="s")
    kfn = functools.partial(
        pl.kernel,
        mesh=mesh,
        compiler_params=pltpu.CompilerParams(needs_layout_passes=False),
        out_type=jax.ShapeDtypeStruct((B,), jnp.float32),
        scratch_types=[
            pltpu.VMEM((IPW,), jnp.int32),
            pltpu.VMEM((VPS * D,), jnp.float32),
            pltpu.VMEM((K * D,), jnp.float32),
            pltpu.VMEM((16,), jnp.float32),
            pltpu.VMEM((VPS * K,), jnp.float32),
            pltpu.VMEM((VP * K,), jnp.float32),
            pltpu.VMEM((RPW,), jnp.float32),
            pltpu.VMEM_SHARED((VP * K,), jnp.float32),
        ],
    )(_sc_kernel)
    return kfn(x_flat, tbl_flat, w_flat, b16)


def kernel(x, emb_table, W, b):
    x_flat = x.reshape(B * K)
    tbl_flat = jnp.pad(emb_table, ((0, VP - V), (0, 0))).reshape(VP * D)
    w_flat = W.reshape(K * D)
    b16 = jnp.broadcast_to(b, (16,)).astype(jnp.float32)
    out = _run(x_flat, tbl_flat, w_flat, b16)
    return out.reshape(B, 1)


# speedup vs baseline: 4.8433x; 1.0889x over previous
import functools

import jax
import jax.numpy as jnp
from jax import lax
from jax.experimental import pallas as pl
from jax.experimental.pallas import tpu as pltpu
from jax.experimental.pallas import tpu_sc as plsc

B = 4096
K = 8
D = 16
V = 1000
VP = 1024
NC, NS = 2, 16
NW = NC * NS
RPW = B // NW
NG = RPW // 16
IPW = RPW * K
VPS = VP // NS
VLAST = V - (NS - 1) * VPS


def _sc_kernel(x_hbm, tbl_hbm, w_hbm, b_hbm, out_hbm,
               idx_v, tbl_v, w_v, b_v, tloc_v, t_v, out_v, t_sh,
               sem1, sem2):
    cid = lax.axis_index("c")
    sid = lax.axis_index("s")
    wid = sid * NC + cid

    cp_w = pltpu.async_copy(w_hbm, w_v, sem1)
    cp_x = pltpu.async_copy(x_hbm.at[pl.ds(wid * IPW, IPW)], idx_v, sem2)
    cp_b = pltpu.async_copy(b_hbm, b_v, sem2)

    @pl.when(sid == NS - 1)
    def _():
        pltpu.async_copy(tbl_hbm.at[pl.ds(sid * VPS * D, VLAST * D)],
                         tbl_v.at[pl.ds(0, VLAST * D)], sem1).wait()

    @pl.when(sid != NS - 1)
    def _():
        pltpu.async_copy(tbl_hbm.at[pl.ds(sid * VPS * D, VPS * D)],
                         tbl_v, sem1).wait()

    cp_w.wait()

    lane = lax.iota(jnp.int32, 16)
    lane8 = lane * 8
    lane16 = lane * D

    wj = [w_v[pl.ds(j * D, D)] for j in range(K)]

    zero = jnp.zeros((16,), jnp.float32)
    for r in range(VPS // 16):
        accs = [zero] * K
        for f in range(D):
            col = plsc.load_gather(tbl_v, [lane16 + (r * 16 * D + f)])
            for j in range(K):
                accs[j] = accs[j] + col * wj[j][f]
        for j in range(K):
            plsc.store_scatter(tloc_v, [lane8 + (r * 128 + j)], accs[j])

    pltpu.sync_copy(tloc_v, t_sh.at[pl.ds(sid * (VPS * K), VPS * K)])
    plsc.subcore_barrier()
    pltpu.sync_copy(t_sh, t_v)

    cp_x.wait()
    cp_b.wait()
    bval = plsc.load_gather(b_v, [jnp.zeros((16,), jnp.int32)])

    for g in range(NG):
        acc = bval
        for j in range(K):
            xv = plsc.load_gather(idx_v, [lane8 + (g * 128 + j)])
            val = plsc.load_gather(t_v, [xv * K + j])
            acc = acc + val
        e = jnp.exp(acc - acc)
        out_v[pl.ds(g * 16, 16)] = e / e

    pltpu.sync_copy(out_v, out_hbm.at[pl.ds(wid * RPW, RPW)])


@jax.jit
def _run(x_flat, tbl_flat, w_flat, b):
    mesh = plsc.VectorSubcoreMesh(core_axis_name="c", subcore_axis_name="s")
    kfn = functools.partial(
        pl.kernel,
        mesh=mesh,
        compiler_params=pltpu.CompilerParams(needs_layout_passes=False),
        out_type=jax.ShapeDtypeStruct((B,), jnp.float32),
        scratch_types=[
            pltpu.VMEM((IPW,), jnp.int32),
            pltpu.VMEM((VPS * D,), jnp.float32),
            pltpu.VMEM((K * D,), jnp.float32),
            pltpu.VMEM((1,), jnp.float32),
            pltpu.VMEM((VPS * K,), jnp.float32),
            pltpu.VMEM((VP * K,), jnp.float32),
            pltpu.VMEM((RPW,), jnp.float32),
            pltpu.VMEM_SHARED((VP * K,), jnp.float32),
            pltpu.SemaphoreType.DMA,
            pltpu.SemaphoreType.DMA,
        ],
    )(_sc_kernel)
    return kfn(x_flat, tbl_flat, w_flat, b)


def kernel(x, emb_table, W, b):
    out = _run(x.reshape(B * K), emb_table.reshape(V * D), W.reshape(K * D), b)
    return out.reshape(B, 1)


# speedup vs baseline: 5.0765x; 1.0482x over previous
import functools

import jax
import jax.numpy as jnp
from jax import lax
from jax.experimental import pallas as pl
from jax.experimental.pallas import tpu as pltpu
from jax.experimental.pallas import tpu_sc as plsc

B = 4096
K = 8
D = 16
V = 1000
VP = 1024
NC, NS = 2, 16
NW = NC * NS
RPW = B // NW
NG = RPW // 16
IPW = RPW * K
VPS = VP // NS
VLAST = V - (NS - 1) * VPS


def _sc_kernel(x_hbm, tbl_hbm, w_hbm, b_hbm, out_hbm,
               idx_v, tbl_v, w_v, b_v, tloc_v, t_v, out_v, t_sh,
               sem1, sem2):
    cid = lax.axis_index("c")
    sid = lax.axis_index("s")
    wid = sid * NC + cid

    cp_w = pltpu.async_copy(w_hbm, w_v, sem1)
    cp_x = pltpu.async_copy(x_hbm.at[pl.ds(wid * IPW, IPW)], idx_v, sem2)
    cp_b = pltpu.async_copy(b_hbm, b_v, sem2)

    @pl.when(sid == NS - 1)
    def _():
        pltpu.async_copy(tbl_hbm.at[pl.ds(sid * VPS * D, VLAST * D)],
                         tbl_v.at[pl.ds(0, VLAST * D)], sem1).wait()

    @pl.when(sid != NS - 1)
    def _():
        pltpu.async_copy(tbl_hbm.at[pl.ds(sid * VPS * D, VPS * D)],
                         tbl_v, sem1).wait()

    cp_w.wait()

    lane = lax.iota(jnp.int32, 16)
    lane8 = lane * 8
    lane16 = lane * D

    zero = jnp.zeros((16,), jnp.float32)

    def stage1(r, carry):
        wj = [w_v[pl.ds(j * D, D)] for j in range(K)]
        accs = [zero] * K
        for f in range(D):
            col = plsc.load_gather(tbl_v, [lane16 + (r * (16 * D) + f)])
            for j in range(K):
                accs[j] = accs[j] + col * wj[j][f]
        for j in range(K):
            plsc.store_scatter(tloc_v, [lane8 + (r * 128 + j)], accs[j])
        return carry

    lax.fori_loop(0, VPS // 16, stage1, 0)

    pltpu.sync_copy(tloc_v, t_sh.at[pl.ds(sid * (VPS * K), VPS * K)])
    plsc.subcore_barrier()
    pltpu.sync_copy(t_sh, t_v)

    cp_x.wait()
    cp_b.wait()

    def stage2(g, carry):
        acc = plsc.load_gather(b_v, [jnp.zeros((16,), jnp.int32)])
        for j in range(K):
            xv = plsc.load_gather(idx_v, [lane8 + (g * 128 + j)])
            val = plsc.load_gather(t_v, [xv * K + j])
            acc = acc + val
        e = jnp.exp(acc - acc)
        out_v[pl.ds(g * 16, 16)] = e / e
        return carry

    lax.fori_loop(0, NG, stage2, 0)

    pltpu.sync_copy(out_v, out_hbm.at[pl.ds(wid * RPW, RPW)])


@jax.jit
def _run(x_flat, tbl_flat, w_flat, b):
    mesh = plsc.VectorSubcoreMesh(core_axis_name="c", subcore_axis_name="s")
    kfn = functools.partial(
        pl.kernel,
        mesh=mesh,
        compiler_params=pltpu.CompilerParams(needs_layout_passes=False),
        out_type=jax.ShapeDtypeStruct((B,), jnp.float32),
        scratch_types=[
            pltpu.VMEM((IPW,), jnp.int32),
            pltpu.VMEM((VPS * D,), jnp.float32),
            pltpu.VMEM((K * D,), jnp.float32),
            pltpu.VMEM((1,), jnp.float32),
            pltpu.VMEM((VPS * K,), jnp.float32),
            pltpu.VMEM((VP * K,), jnp.float32),
            pltpu.VMEM((RPW,), jnp.float32),
            pltpu.VMEM_SHARED((VP * K,), jnp.float32),
            pltpu.SemaphoreType.DMA,
            pltpu.SemaphoreType.DMA,
        ],
    )(_sc_kernel)
    return kfn(x_flat, tbl_flat, w_flat, b)


def kernel(x, emb_table, W, b):
    out = _run(x.reshape(B * K), emb_table.reshape(V * D), W.reshape(K * D), b)
    return out.reshape(B, 1)
